# Initial kernel scaffold; baseline (speedup 1.0000x reference)
import jax
import jax.numpy as jnp
from jax.experimental import pallas as pl


def kernel(out, x1, x2, x3, x4, W1, b1, W2, b2, Wm, bm):
    raise NotImplementedError("write your pallas kernel here")


# speedup vs baseline: 8.2150x; 8.2150x over previous
import jax
import jax.numpy as jnp
from jax.experimental import pallas as pl
from jax.experimental.pallas import tpu as pltpu

_CONF = 0.596
_IOU = 0.45
_KMAX = 200
_N = 20000
_NPAD = 20480
_ROWS = 160
_LANES = 128
_NEG = float("-inf")


def _prep_kernel(x_ref, sc_ref, boff_ref, braw_ref, area_ref):
    x = x_ref[...]
    obj = x[:, 4:5]
    cs = x[:, 5:85] * obj
    conf = jnp.max(cs, axis=1, keepdims=True)
    lane = jax.lax.broadcasted_iota(jnp.int32, cs.shape, 1)
    jm = jnp.min(jnp.where(cs == conf, lane, 127), axis=1, keepdims=True)
    valid = (obj > _CONF) & (conf > _CONF)
    sc_ref[...] = jnp.where(valid, conf, _NEG)
    xy = x[:, 0:2]
    half = x[:, 2:4] * 0.5
    braw = jnp.concatenate([xy - half, xy + half], axis=1)
    braw_ref[...] = braw
    boff = braw + jm.astype(jnp.float32) * 4096.0
    boff_ref[...] = boff
    area_ref[...] = (boff[:, 2:3] - boff[:, 0:1]) * (boff[:, 3:4] - boff[:, 1:2])


def _nms_kernel(sc_ref, bx1_ref, by1_ref, bx2_ref, by2_ref, ar_ref,
                rx1_ref, ry1_ref, rx2_ref, ry2_ref, keep_ref, kept_ref):
    bx1 = bx1_ref[...]
    by1 = by1_ref[...]
    bx2 = bx2_ref[...]
    by2 = by2_ref[...]
    area = ar_ref[...]
    lin = (jax.lax.broadcasted_iota(jnp.int32, (_ROWS, _LANES), 0) * _LANES
           + jax.lax.broadcasted_iota(jnp.int32, (_ROWS, _LANES), 1))
    lane1 = jax.lax.broadcasted_iota(jnp.int32, (1, _LANES), 1)

    def body(k, score):
        m = jnp.max(score)
        i = jnp.min(jnp.where(score == m, lin, jnp.int32(2**30)))
        keep_ref[k] = i
        r = i // _LANES
        c = i - r * _LANES
        ohc = lane1 == c

        def pick(ref):
            rowv = ref[pl.ds(r, 1), :]
            return jnp.sum(jnp.where(ohc, rowv, 0.0), axis=1, keepdims=True)

        kept_ref[pl.ds(k, 1), :] = jnp.concatenate(
            [pick(rx1_ref), pick(ry1_ref), pick(rx2_ref), pick(ry2_ref)],
            axis=1)
        xx1 = jnp.maximum(bx1, pick(bx1_ref))
        yy1 = jnp.maximum(by1, pick(by1_ref))
        xx2 = jnp.minimum(bx2, pick(bx2_ref))
        yy2 = jnp.minimum(by2, pick(by2_ref))
        inter = jnp.maximum(xx2 - xx1, 0.0) * jnp.maximum(yy2 - yy1, 0.0)
        denom = ((pick(ar_ref) + area) - inter) + 1e-9
        kill = (inter > _IOU * denom) | (lin == i)
        return jnp.where(kill, _NEG, score)

    jax.lax.fori_loop(0, _KMAX, body, sc_ref[...])


def _head_kernel(kept_ref, fm1_ref, fm2_ref, fm3_ref, fm4_ref,
                 w1_ref, c1_ref, w2_ref, c2_ref, wm_ref, cm_ref,
                 res_ref, cell_ref):
    bbox = kept_ref[...]

    def _leaky(v):
        return jnp.where(v >= 0, v, 0.01 * v)

    def roi(fm_t, scale, h, w):
        b = bbox * scale
        x1 = b[:, 0:1]
        y1 = b[:, 1:2]
        rw = jnp.maximum(b[:, 2:3] - x1, 1.0)
        rh = jnp.maximum(b[:, 3:4] - y1, 1.0)
        sx0 = x1 + 0.25 * rw
        sx1 = x1 + 0.75 * rw
        sy0 = y1 + 0.25 * rh
        sy1 = y1 + 0.75 * rh
        yy = jnp.concatenate([sy0, sy0, sy1, sy1], axis=1)
        xx = jnp.concatenate([sx0, sx1, sx0, sx1], axis=1)
        y = jnp.clip(yy, 0.0, h - 1.0)
        x = jnp.clip(xx, 0.0, w - 1.0)
        y0 = jnp.floor(y)
        x0 = jnp.floor(x)
        y0i = y0.astype(jnp.int32)
        x0i = x0.astype(jnp.int32)
        y1i = jnp.minimum(y0i + 1, h - 1)
        x1i = jnp.minimum(x0i + 1, w - 1)
        ly = y - y0
        lx = x - x0
        hy = 1.0 - ly
        hx = 1.0 - lx
        idxs = jnp.concatenate([y0i * w + x0i, y0i * w + x1i,
                                y1i * w + x0i, y1i * w + x1i], axis=1)
        ws = jnp.concatenate([hy * hx, hy * lx, ly * hx, ly * lx],
                             axis=1) * 0.25
        hw = h * w
        pos = jax.lax.broadcasted_iota(jnp.int32, (256, hw), 1)
        s = jnp.zeros((256, hw), jnp.float32)
        for t in range(16):
            s = s + jnp.where(pos == idxs[:, t:t + 1], ws[:, t:t + 1], 0.0)
        return jnp.dot(s, fm_t, preferred_element_type=jnp.float32)

    f1 = roi(fm1_ref[...], 1.0 / 8, 64, 64)
    f2 = roi(fm2_ref[...], 1.0 / 16, 32, 32)
    f3 = roi(fm3_ref[...], 1.0 / 32, 16, 16)
    f4 = roi(fm4_ref[...], 1.0 / 64, 8, 8)
    w1 = w1_ref[...]
    h = (jnp.dot(f1, w1[0:128], preferred_element_type=jnp.float32)
         + jnp.dot(f2, w1[128:384], preferred_element_type=jnp.float32)
         + jnp.dot(f3, w1[384:896], preferred_element_type=jnp.float32)
         + jnp.dot(f4, w1[896:1920], preferred_element_type=jnp.float32)
         + c1_ref[...])
    h = _leaky(h)
    h = _leaky(jnp.dot(h, w2_ref[...], preferred_element_type=jnp.float32)
               + c2_ref[...])
    res = jnp.concatenate([bbox * (1.0 / 512.0), h], axis=1)
    cells = jnp.dot(res, wm_ref[...], preferred_element_type=jnp.float32) \
        + cm_ref[...]
    res_ref[...] = res[0:_KMAX]
    cell_ref[...] = cells[0:_KMAX]


def kernel(out, x1, x2, x3, x4, W1, b1, W2, b2, Wm, bm):
    o = out[0]
    xp = jnp.pad(o, ((0, _NPAD - _N), (0, 0)))
    nblk = 20
    blk = _NPAD // nblk
    sc, boff, braw, area = pl.pallas_call(
        _prep_kernel,
        grid=(nblk,),
        in_specs=[pl.BlockSpec((blk, 85), lambda g: (g, 0))],
        out_specs=[
            pl.BlockSpec((blk, 1), lambda g: (g, 0)),
            pl.BlockSpec((blk, 4), lambda g: (g, 0)),
            pl.BlockSpec((blk, 4), lambda g: (g, 0)),
            pl.BlockSpec((blk, 1), lambda g: (g, 0)),
        ],
        out_shape=[
            jax.ShapeDtypeStruct((_NPAD, 1), jnp.float32),
            jax.ShapeDtypeStruct((_NPAD, 4), jnp.float32),
            jax.ShapeDtypeStruct((_NPAD, 4), jnp.float32),
            jax.ShapeDtypeStruct((_NPAD, 1), jnp.float32),
        ],
    )(xp)
    sc2 = sc.reshape(_ROWS, _LANES)
    bx1 = boff[:, 0].reshape(_ROWS, _LANES)
    by1 = boff[:, 1].reshape(_ROWS, _LANES)
    bx2 = boff[:, 2].reshape(_ROWS, _LANES)
    by2 = boff[:, 3].reshape(_ROWS, _LANES)
    rx1 = braw[:, 0].reshape(_ROWS, _LANES)
    ry1 = braw[:, 1].reshape(_ROWS, _LANES)
    rx2 = braw[:, 2].reshape(_ROWS, _LANES)
    ry2 = braw[:, 3].reshape(_ROWS, _LANES)
    ar2 = area.reshape(_ROWS, _LANES)
    _, kept = pl.pallas_call(
        _nms_kernel,
        out_shape=[
            jax.ShapeDtypeStruct((256,), jnp.int32),
            jax.ShapeDtypeStruct((256, 4), jnp.float32),
        ],
        out_specs=[
            pl.BlockSpec(memory_space=pltpu.SMEM),
            pl.BlockSpec(memory_space=pltpu.VMEM),
        ],
    )(sc2, bx1, by1, bx2, by2, ar2, rx1, ry1, rx2, ry2)
    fm1 = jnp.transpose(x1[0].reshape(128, 4096))
    fm2 = jnp.transpose(x2[0].reshape(256, 1024))
    fm3 = jnp.transpose(x3[0].reshape(512, 256))
    fm4 = jnp.transpose(x4[0].reshape(1024, 64))
    result, cells = pl.pallas_call(
        _head_kernel,
        out_shape=[
            jax.ShapeDtypeStruct((_KMAX, 68), jnp.float32),
            jax.ShapeDtypeStruct((_KMAX, 2), jnp.float32),
        ],
    )(kept, fm1, fm2, fm3, fm4,
      W1, b1.reshape(1, 64), W2, b2.reshape(1, 64), Wm, bm.reshape(1, 2))
    return out, result, cells


# speedup vs baseline: 8.3168x; 1.0124x over previous
import jax
import jax.numpy as jnp
from jax.experimental import pallas as pl
from jax.experimental.pallas import tpu as pltpu

_CONF = 0.596
_IOU = 0.45
_KMAX = 200
_N = 20000
_NPAD = 20480
_ROWS = 160
_LANES = 128
_NEG = float("-inf")


def _prep_kernel(x_ref, sc_ref, boff_ref, braw_ref, area_ref):
    x = x_ref[...]
    obj = x[:, 4:5]
    cs = x[:, 5:85] * obj
    conf = jnp.max(cs, axis=1, keepdims=True)
    lane = jax.lax.broadcasted_iota(jnp.int32, cs.shape, 1)
    jm = jnp.min(jnp.where(cs == conf, lane, 127), axis=1, keepdims=True)
    row = (pl.program_id(0) * (x.shape[0])
           + jax.lax.broadcasted_iota(jnp.int32, (x.shape[0], 1), 0))
    valid = (obj > _CONF) & (conf > _CONF) & (row < _N)
    sc_ref[...] = jnp.where(valid, conf, _NEG)
    xy = x[:, 0:2]
    half = x[:, 2:4] * 0.5
    braw = jnp.concatenate([xy - half, xy + half], axis=1)
    braw_ref[...] = braw
    boff = braw + jm.astype(jnp.float32) * 4096.0
    boff_ref[...] = boff
    area_ref[...] = (boff[:, 2:3] - boff[:, 0:1]) * (boff[:, 3:4] - boff[:, 1:2])


def _nms_kernel(sc_ref, bx1_ref, by1_ref, bx2_ref, by2_ref, ar_ref,
                rx1_ref, ry1_ref, rx2_ref, ry2_ref, keep_ref, kept_ref):
    bx1 = bx1_ref[...]
    by1 = by1_ref[...]
    bx2 = bx2_ref[...]
    by2 = by2_ref[...]
    area = ar_ref[...]
    lin = (jax.lax.broadcasted_iota(jnp.int32, (_ROWS, _LANES), 0) * _LANES
           + jax.lax.broadcasted_iota(jnp.int32, (_ROWS, _LANES), 1))
    lane1 = jax.lax.broadcasted_iota(jnp.int32, (1, _LANES), 1)

    def body(k, score):
        m = jnp.max(score)
        i = jnp.min(jnp.where(score == m, lin, jnp.int32(2**30)))
        keep_ref[k] = i
        r = i // _LANES
        c = i - r * _LANES
        ohc = lane1 == c

        def pick(ref):
            rowv = ref[pl.ds(r, 1), :]
            return jnp.sum(jnp.where(ohc, rowv, 0.0), axis=1, keepdims=True)

        kept_ref[pl.ds(k, 1), :] = jnp.concatenate(
            [pick(rx1_ref), pick(ry1_ref), pick(rx2_ref), pick(ry2_ref)],
            axis=1)
        xx1 = jnp.maximum(bx1, pick(bx1_ref))
        yy1 = jnp.maximum(by1, pick(by1_ref))
        xx2 = jnp.minimum(bx2, pick(bx2_ref))
        yy2 = jnp.minimum(by2, pick(by2_ref))
        inter = jnp.maximum(xx2 - xx1, 0.0) * jnp.maximum(yy2 - yy1, 0.0)
        denom = ((pick(ar_ref) + area) - inter) + 1e-9
        return jnp.where(inter > _IOU * denom, _NEG, score)

    jax.lax.fori_loop(0, _KMAX, body, sc_ref[...])


def _head_kernel(kept_ref, fm1_ref, fm2_ref, fm3_ref, fm4_ref,
                 w1_ref, c1_ref, w2_ref, c2_ref, wm_ref, cm_ref,
                 res_ref, cell_ref):
    bbox = kept_ref[...]

    def _leaky(v):
        return jnp.where(v >= 0, v, 0.01 * v)

    def roi(fm_t, scale, h, w):
        b = bbox * scale
        x1 = b[:, 0:1]
        y1 = b[:, 1:2]
        rw = jnp.maximum(b[:, 2:3] - x1, 1.0)
        rh = jnp.maximum(b[:, 3:4] - y1, 1.0)
        sx0 = x1 + 0.25 * rw
        sx1 = x1 + 0.75 * rw
        sy0 = y1 + 0.25 * rh
        sy1 = y1 + 0.75 * rh
        yy = jnp.concatenate([sy0, sy0, sy1, sy1], axis=1)
        xx = jnp.concatenate([sx0, sx1, sx0, sx1], axis=1)
        y = jnp.clip(yy, 0.0, h - 1.0)
        x = jnp.clip(xx, 0.0, w - 1.0)
        y0 = jnp.floor(y)
        x0 = jnp.floor(x)
        y0i = y0.astype(jnp.int32)
        x0i = x0.astype(jnp.int32)
        y1i = jnp.minimum(y0i + 1, h - 1)
        x1i = jnp.minimum(x0i + 1, w - 1)
        ly = y - y0
        lx = x - x0
        hy = 1.0 - ly
        hx = 1.0 - lx
        idxs = jnp.concatenate([y0i * w + x0i, y0i * w + x1i,
                                y1i * w + x0i, y1i * w + x1i], axis=1)
        ws = jnp.concatenate([hy * hx, hy * lx, ly * hx, ly * lx],
                             axis=1) * 0.25
        hw = h * w
        pos = jax.lax.broadcasted_iota(jnp.int32, (256, hw), 1)
        s = jnp.zeros((256, hw), jnp.float32)
        for t in range(16):
            s = s + jnp.where(pos == idxs[:, t:t + 1], ws[:, t:t + 1], 0.0)
        return jax.lax.dot_general(
            s, fm_t, dimension_numbers=(((1,), (1,)), ((), ())),
            preferred_element_type=jnp.float32,
            precision=jax.lax.Precision.HIGHEST)

    def dot(a, b):
        return jnp.dot(a, b, preferred_element_type=jnp.float32,
                       precision=jax.lax.Precision.HIGHEST)

    f1 = roi(fm1_ref[...], 1.0 / 8, 64, 64)
    f2 = roi(fm2_ref[...], 1.0 / 16, 32, 32)
    f3 = roi(fm3_ref[...], 1.0 / 32, 16, 16)
    f4 = roi(fm4_ref[...], 1.0 / 64, 8, 8)
    w1 = w1_ref[...]
    h = (dot(f1, w1[0:128]) + dot(f2, w1[128:384]) + dot(f3, w1[384:896])
         + dot(f4, w1[896:1920]) + c1_ref[...])
    h = _leaky(h)
    h = _leaky(dot(h, w2_ref[...]) + c2_ref[...])
    res = jnp.concatenate([bbox * (1.0 / 512.0), h], axis=1)
    cells = dot(res, wm_ref[...]) + cm_ref[...]
    res_ref[...] = res[0:_KMAX]
    cell_ref[...] = cells[0:_KMAX]


def kernel(out, x1, x2, x3, x4, W1, b1, W2, b2, Wm, bm):
    o = out[0]
    nblk = 20
    blk = _NPAD // nblk
    sc, boff, braw, area = pl.pallas_call(
        _prep_kernel,
        grid=(nblk,),
        in_specs=[pl.BlockSpec((blk, 85), lambda g: (g, 0))],
        out_specs=[
            pl.BlockSpec((blk, 1), lambda g: (g, 0)),
            pl.BlockSpec((blk, 4), lambda g: (g, 0)),
            pl.BlockSpec((blk, 4), lambda g: (g, 0)),
            pl.BlockSpec((blk, 1), lambda g: (g, 0)),
        ],
        out_shape=[
            jax.ShapeDtypeStruct((_NPAD, 1), jnp.float32),
            jax.ShapeDtypeStruct((_NPAD, 4), jnp.float32),
            jax.ShapeDtypeStruct((_NPAD, 4), jnp.float32),
            jax.ShapeDtypeStruct((_NPAD, 1), jnp.float32),
        ],
    )(o)
    sc2 = sc.reshape(_ROWS, _LANES)
    bx1 = boff[:, 0].reshape(_ROWS, _LANES)
    by1 = boff[:, 1].reshape(_ROWS, _LANES)
    bx2 = boff[:, 2].reshape(_ROWS, _LANES)
    by2 = boff[:, 3].reshape(_ROWS, _LANES)
    rx1 = braw[:, 0].reshape(_ROWS, _LANES)
    ry1 = braw[:, 1].reshape(_ROWS, _LANES)
    rx2 = braw[:, 2].reshape(_ROWS, _LANES)
    ry2 = braw[:, 3].reshape(_ROWS, _LANES)
    ar2 = area.reshape(_ROWS, _LANES)
    _, kept = pl.pallas_call(
        _nms_kernel,
        out_shape=[
            jax.ShapeDtypeStruct((256,), jnp.int32),
            jax.ShapeDtypeStruct((256, 4), jnp.float32),
        ],
        out_specs=[
            pl.BlockSpec(memory_space=pltpu.SMEM),
            pl.BlockSpec(memory_space=pltpu.VMEM),
        ],
    )(sc2, bx1, by1, bx2, by2, ar2, rx1, ry1, rx2, ry2)
    fm1 = x1[0].reshape(128, 4096)
    fm2 = x2[0].reshape(256, 1024)
    fm3 = x3[0].reshape(512, 256)
    fm4 = x4[0].reshape(1024, 64)
    result, cells = pl.pallas_call(
        _head_kernel,
        out_shape=[
            jax.ShapeDtypeStruct((_KMAX, 68), jnp.float32),
            jax.ShapeDtypeStruct((_KMAX, 2), jnp.float32),
        ],
    )(kept, fm1, fm2, fm3, fm4,
      W1, b1.reshape(1, 64), W2, b2.reshape(1, 64), Wm, bm.reshape(1, 2))
    return out, result, cells


# speedup vs baseline: 8.6556x; 1.0407x over previous
import jax
import jax.numpy as jnp
from jax.experimental import pallas as pl
from jax.experimental.pallas import tpu as pltpu

_CONF = 0.596
_IOU = 0.45
_KMAX = 200
_N = 20000
_NPAD = 20480
_ROWS = 160
_LANES = 128
_NEG = float("-inf")


def _prep_kernel(x_ref, sc_ref, boff_ref, braw_ref):
    x = x_ref[...]
    obj = x[:, 4:5]
    cs = x[:, 5:85] * obj
    conf = jnp.max(cs, axis=1, keepdims=True)
    lane = jax.lax.broadcasted_iota(jnp.int32, cs.shape, 1)
    jm = jnp.min(jnp.where(cs == conf, lane, 127), axis=1, keepdims=True)
    off = jm.astype(jnp.float32) * 4096.0
    row = (pl.program_id(0) * (x.shape[0])
           + jax.lax.broadcasted_iota(jnp.int32, (x.shape[0], 1), 0))
    valid = (obj > _CONF) & (conf > _CONF) & (row < _N)
    sc_ref[...] = jnp.where(valid, conf, _NEG)
    xy = x[:, 0:2]
    half = x[:, 2:4] * 0.5
    braw = jnp.concatenate([xy - half, xy + half], axis=1)
    braw_ref[...] = braw
    boff_ref[...] = braw + off


def _nms_kernel(sc_ref, bx1_ref, by1_ref, bx2_ref, by2_ref, pk_ref, kept_ref):
    bx1 = bx1_ref[...]
    by1 = by1_ref[...]
    bx2 = bx2_ref[...]
    by2 = by2_ref[...]
    area = (bx2 - bx1) * (by2 - by1)
    lin = (jax.lax.broadcasted_iota(jnp.int32, (_ROWS, _LANES), 0) * _LANES
           + jax.lax.broadcasted_iota(jnp.int32, (_ROWS, _LANES), 1))
    lane1 = jax.lax.broadcasted_iota(jnp.int32, (1, _LANES), 1)

    def body(k, score):
        m = jnp.max(score)
        i = jnp.min(jnp.where(score == m, lin, jnp.int32(2**30)))
        r = i // _LANES
        c = i - r * _LANES
        rows = pk_ref[pl.ds(r, 1), :, :].reshape(8, _LANES)
        vals = jnp.sum(jnp.where(lane1 == c, rows, 0.0),
                       axis=1, keepdims=True)
        x1i = vals[0:1, 0:1]
        y1i = vals[1:2, 0:1]
        x2i = vals[2:3, 0:1]
        y2i = vals[3:4, 0:1]
        kept_ref[pl.ds(k, 1), :] = jnp.concatenate(
            [vals[4:5, 0:1], vals[5:6, 0:1], vals[6:7, 0:1], vals[7:8, 0:1]],
            axis=1)
        ai = (x2i - x1i) * (y2i - y1i)
        xx1 = jnp.maximum(bx1, x1i)
        yy1 = jnp.maximum(by1, y1i)
        xx2 = jnp.minimum(bx2, x2i)
        yy2 = jnp.minimum(by2, y2i)
        inter = jnp.maximum(xx2 - xx1, 0.0) * jnp.maximum(yy2 - yy1, 0.0)
        denom = ((ai + area) - inter) + 1e-9
        return jnp.where(inter > _IOU * denom, _NEG, score)

    jax.lax.fori_loop(0, _KMAX, body, sc_ref[...])


def _head_kernel(kept_ref, fm1_ref, fm2_ref, fm3_ref, fm4_ref,
                 w1_ref, c1_ref, w2_ref, c2_ref, wm_ref, cm_ref,
                 res_ref, cell_ref):
    bbox = kept_ref[...]

    def _leaky(v):
        return jnp.where(v >= 0, v, 0.01 * v)

    def roi(fm_t, scale, h, w):
        b = bbox * scale
        x1 = b[:, 0:1]
        y1 = b[:, 1:2]
        rw = jnp.maximum(b[:, 2:3] - x1, 1.0)
        rh = jnp.maximum(b[:, 3:4] - y1, 1.0)
        sx0 = x1 + 0.25 * rw
        sx1 = x1 + 0.75 * rw
        sy0 = y1 + 0.25 * rh
        sy1 = y1 + 0.75 * rh
        yy = jnp.concatenate([sy0, sy0, sy1, sy1], axis=1)
        xx = jnp.concatenate([sx0, sx1, sx0, sx1], axis=1)
        y = jnp.clip(yy, 0.0, h - 1.0)
        x = jnp.clip(xx, 0.0, w - 1.0)
        y0 = jnp.floor(y)
        x0 = jnp.floor(x)
        y0i = y0.astype(jnp.int32)
        x0i = x0.astype(jnp.int32)
        y1i = jnp.minimum(y0i + 1, h - 1)
        x1i = jnp.minimum(x0i + 1, w - 1)
        ly = y - y0
        lx = x - x0
        hy = 1.0 - ly
        hx = 1.0 - lx
        idxs = jnp.concatenate([y0i * w + x0i, y0i * w + x1i,
                                y1i * w + x0i, y1i * w + x1i], axis=1)
        ws = jnp.concatenate([hy * hx, hy * lx, ly * hx, ly * lx],
                             axis=1) * 0.25
        hw = h * w
        pos = jax.lax.broadcasted_iota(jnp.int32, (256, hw), 1)
        s = jnp.zeros((256, hw), jnp.float32)
        for t in range(16):
            s = s + jnp.where(pos == idxs[:, t:t + 1], ws[:, t:t + 1], 0.0)
        return jax.lax.dot_general(
            s, fm_t, dimension_numbers=(((1,), (1,)), ((), ())),
            preferred_element_type=jnp.float32,
            precision=jax.lax.Precision.HIGHEST)

    def dot(a, b):
        return jnp.dot(a, b, preferred_element_type=jnp.float32,
                       precision=jax.lax.Precision.HIGHEST)

    f1 = roi(fm1_ref[...], 1.0 / 8, 64, 64)
    f2 = roi(fm2_ref[...], 1.0 / 16, 32, 32)
    f3 = roi(fm3_ref[...], 1.0 / 32, 16, 16)
    f4 = roi(fm4_ref[...], 1.0 / 64, 8, 8)
    w1 = w1_ref[...]
    h = (dot(f1, w1[0:128]) + dot(f2, w1[128:384]) + dot(f3, w1[384:896])
         + dot(f4, w1[896:1920]) + c1_ref[...])
    h = _leaky(h)
    h = _leaky(dot(h, w2_ref[...]) + c2_ref[...])
    res = jnp.concatenate([bbox * (1.0 / 512.0), h], axis=1)
    cells = dot(res, wm_ref[...]) + cm_ref[...]
    res_ref[...] = res[0:_KMAX]
    cell_ref[...] = cells[0:_KMAX]


def kernel(out, x1, x2, x3, x4, W1, b1, W2, b2, Wm, bm):
    o = out[0]
    nblk = 20
    blk = _NPAD // nblk
    sc, boff, braw = pl.pallas_call(
        _prep_kernel,
        grid=(nblk,),
        in_specs=[pl.BlockSpec((blk, 85), lambda g: (g, 0))],
        out_specs=[
            pl.BlockSpec((blk, 1), lambda g: (g, 0)),
            pl.BlockSpec((blk, 4), lambda g: (g, 0)),
            pl.BlockSpec((blk, 4), lambda g: (g, 0)),
        ],
        out_shape=[
            jax.ShapeDtypeStruct((_NPAD, 1), jnp.float32),
            jax.ShapeDtypeStruct((_NPAD, 4), jnp.float32),
            jax.ShapeDtypeStruct((_NPAD, 4), jnp.float32),
        ],
    )(o)
    sc2 = sc.reshape(_ROWS, _LANES)
    bx1 = boff[:, 0].reshape(_ROWS, _LANES)
    by1 = boff[:, 1].reshape(_ROWS, _LANES)
    bx2 = boff[:, 2].reshape(_ROWS, _LANES)
    by2 = boff[:, 3].reshape(_ROWS, _LANES)
    rx1 = braw[:, 0].reshape(_ROWS, _LANES)
    ry1 = braw[:, 1].reshape(_ROWS, _LANES)
    rx2 = braw[:, 2].reshape(_ROWS, _LANES)
    ry2 = braw[:, 3].reshape(_ROWS, _LANES)
    packed = jnp.stack([bx1, by1, bx2, by2, rx1, ry1, rx2, ry2], axis=1)
    kept = pl.pallas_call(
        _nms_kernel,
        out_shape=jax.ShapeDtypeStruct((256, 4), jnp.float32),
    )(sc2, bx1, by1, bx2, by2, packed)
    fm1 = x1[0].reshape(128, 4096)
    fm2 = x2[0].reshape(256, 1024)
    fm3 = x3[0].reshape(512, 256)
    fm4 = x4[0].reshape(1024, 64)
    result, cells = pl.pallas_call(
        _head_kernel,
        out_shape=[
            jax.ShapeDtypeStruct((_KMAX, 68), jnp.float32),
            jax.ShapeDtypeStruct((_KMAX, 2), jnp.float32),
        ],
    )(kept, fm1, fm2, fm3, fm4,
      W1, b1.reshape(1, 64), W2, b2.reshape(1, 64), Wm, bm.reshape(1, 2))
    return out, result, cells


# speedup vs baseline: 9.9414x; 1.1485x over previous
import jax
import jax.numpy as jnp
from jax.experimental import pallas as pl
from jax.experimental.pallas import tpu as pltpu

_CONF = 0.596
_IOU = 0.45
_KMAX = 200
_N = 20000
_NPAD = 20480
_ROWS = 160
_LANES = 128
_NEG = float("-inf")


def _prep_kernel(x_ref, sc_ref, bx1_ref, by1_ref, bx2_ref, by2_ref, pk_ref):
    x = x_ref[...]
    obj = x[:, 4:5]
    cs = x[:, 5:85] * obj
    conf = jnp.max(cs, axis=1, keepdims=True)
    lane = jax.lax.broadcasted_iota(jnp.int32, cs.shape, 1)
    jm = jnp.min(jnp.where(cs == conf, lane, 127), axis=1, keepdims=True)
    off = jm.astype(jnp.float32) * 4096.0
    row = (pl.program_id(0) * (x.shape[0])
           + jax.lax.broadcasted_iota(jnp.int32, (x.shape[0], 1), 0))
    valid = (obj > _CONF) & (conf > _CONF) & (row < _N)
    sc_ref[...] = jnp.where(valid, conf, _NEG).reshape(8, _LANES)
    xy = x[:, 0:2]
    half = x[:, 2:4] * 0.5
    braw = jnp.concatenate([xy - half, xy + half], axis=1)
    boff = braw + off
    cols = [boff[:, t:t + 1].reshape(8, _LANES) for t in range(4)]
    cols += [braw[:, t:t + 1].reshape(8, _LANES) for t in range(4)]
    bx1_ref[...] = cols[0]
    by1_ref[...] = cols[1]
    bx2_ref[...] = cols[2]
    by2_ref[...] = cols[3]
    pk_ref[...] = jnp.stack(cols, axis=1)


def _nms_head_kernel(sc_ref, bx1_ref, by1_ref, bx2_ref, by2_ref, pk_ref,
                     fm1_ref, fm2_ref, fm3_ref, fm4_ref,
                     w1_ref, c1_ref, w2_ref, c2_ref, wm_ref, cm_ref,
                     res_ref, cell_ref, kept_s):
    bx1 = bx1_ref[...]
    by1 = by1_ref[...]
    bx2 = bx2_ref[...]
    by2 = by2_ref[...]
    area = (bx2 - bx1) * (by2 - by1)
    lin = (jax.lax.broadcasted_iota(jnp.int32, (_ROWS, _LANES), 0) * _LANES
           + jax.lax.broadcasted_iota(jnp.int32, (_ROWS, _LANES), 1))
    lane1 = jax.lax.broadcasted_iota(jnp.int32, (1, _LANES), 1)
    kept_s[...] = jnp.zeros((256, 4), jnp.float32)

    def body(k, score):
        m = jnp.max(score)
        i = jnp.min(jnp.where(score == m, lin, jnp.int32(2**30)))
        r = i // _LANES
        c = i - r * _LANES
        rows = pk_ref[pl.ds(r, 1), :, :].reshape(8, _LANES)
        vals = jnp.sum(jnp.where(lane1 == c, rows, 0.0),
                       axis=1, keepdims=True)
        x1i = vals[0:1, 0:1]
        y1i = vals[1:2, 0:1]
        x2i = vals[2:3, 0:1]
        y2i = vals[3:4, 0:1]
        kept_s[pl.ds(k, 1), :] = jnp.concatenate(
            [vals[4:5, 0:1], vals[5:6, 0:1], vals[6:7, 0:1], vals[7:8, 0:1]],
            axis=1)
        ai = (x2i - x1i) * (y2i - y1i)
        xx1 = jnp.maximum(bx1, x1i)
        yy1 = jnp.maximum(by1, y1i)
        xx2 = jnp.minimum(bx2, x2i)
        yy2 = jnp.minimum(by2, y2i)
        inter = jnp.maximum(xx2 - xx1, 0.0) * jnp.maximum(yy2 - yy1, 0.0)
        denom = ((ai + area) - inter) + 1e-9
        return jnp.where(inter > _IOU * denom, _NEG, score)

    jax.lax.fori_loop(0, _KMAX, body, sc_ref[...])
    bbox = kept_s[...]

    def _leaky(v):
        return jnp.where(v >= 0, v, 0.01 * v)

    def roi(fm_t, scale, h, w):
        b = bbox * scale
        x1 = b[:, 0:1]
        y1 = b[:, 1:2]
        rw = jnp.maximum(b[:, 2:3] - x1, 1.0)
        rh = jnp.maximum(b[:, 3:4] - y1, 1.0)
        sx0 = x1 + 0.25 * rw
        sx1 = x1 + 0.75 * rw
        sy0 = y1 + 0.25 * rh
        sy1 = y1 + 0.75 * rh
        yy = jnp.concatenate([sy0, sy0, sy1, sy1], axis=1)
        xx = jnp.concatenate([sx0, sx1, sx0, sx1], axis=1)
        y = jnp.clip(yy, 0.0, h - 1.0)
        x = jnp.clip(xx, 0.0, w - 1.0)
        y0 = jnp.floor(y)
        x0 = jnp.floor(x)
        y0i = y0.astype(jnp.int32)
        x0i = x0.astype(jnp.int32)
        y1i = jnp.minimum(y0i + 1, h - 1)
        x1i = jnp.minimum(x0i + 1, w - 1)
        ly = y - y0
        lx = x - x0
        hy = 1.0 - ly
        hx = 1.0 - lx
        idxs = jnp.concatenate([y0i * w + x0i, y0i * w + x1i,
                                y1i * w + x0i, y1i * w + x1i], axis=1)
        ws = jnp.concatenate([hy * hx, hy * lx, ly * hx, ly * lx],
                             axis=1) * 0.25
        hw = h * w
        pos = jax.lax.broadcasted_iota(jnp.int32, (256, hw), 1)
        s = jnp.zeros((256, hw), jnp.float32)
        for t in range(16):
            s = s + jnp.where(pos == idxs[:, t:t + 1], ws[:, t:t + 1], 0.0)
        return jax.lax.dot_general(
            s, fm_t, dimension_numbers=(((1,), (1,)), ((), ())),
            preferred_element_type=jnp.float32,
            precision=jax.lax.Precision.HIGHEST)

    def dot(a, b):
        return jnp.dot(a, b, preferred_element_type=jnp.float32,
                       precision=jax.lax.Precision.HIGHEST)

    f1 = roi(fm1_ref[...], 1.0 / 8, 64, 64)
    f2 = roi(fm2_ref[...], 1.0 / 16, 32, 32)
    f3 = roi(fm3_ref[...], 1.0 / 32, 16, 16)
    f4 = roi(fm4_ref[...], 1.0 / 64, 8, 8)
    w1 = w1_ref[...]
    h = (dot(f1, w1[0:128]) + dot(f2, w1[128:384]) + dot(f3, w1[384:896])
         + dot(f4, w1[896:1920]) + c1_ref[...])
    h = _leaky(h)
    h = _leaky(dot(h, w2_ref[...]) + c2_ref[...])
    res = jnp.concatenate([bbox * (1.0 / 512.0), h], axis=1)
    cells = dot(res, wm_ref[...]) + cm_ref[...]
    res_ref[...] = res[0:_KMAX]
    cell_ref[...] = cells[0:_KMAX]


def kernel(out, x1, x2, x3, x4, W1, b1, W2, b2, Wm, bm):
    o = out[0]
    nblk = 20
    blk = _NPAD // nblk
    sc2, bx1, by1, bx2, by2, packed = pl.pallas_call(
        _prep_kernel,
        grid=(nblk,),
        in_specs=[pl.BlockSpec((blk, 85), lambda g: (g, 0))],
        out_specs=[
            pl.BlockSpec((8, _LANES), lambda g: (g, 0)),
            pl.BlockSpec((8, _LANES), lambda g: (g, 0)),
            pl.BlockSpec((8, _LANES), lambda g: (g, 0)),
            pl.BlockSpec((8, _LANES), lambda g: (g, 0)),
            pl.BlockSpec((8, _LANES), lambda g: (g, 0)),
            pl.BlockSpec((8, 8, _LANES), lambda g: (g, 0, 0)),
        ],
        out_shape=[
            jax.ShapeDtypeStruct((_ROWS, _LANES), jnp.float32),
            jax.ShapeDtypeStruct((_ROWS, _LANES), jnp.float32),
            jax.ShapeDtypeStruct((_ROWS, _LANES), jnp.float32),
            jax.ShapeDtypeStruct((_ROWS, _LANES), jnp.float32),
            jax.ShapeDtypeStruct((_ROWS, _LANES), jnp.float32),
            jax.ShapeDtypeStruct((_ROWS, 8, _LANES), jnp.float32),
        ],
    )(o)
    result, cells = pl.pallas_call(
        _nms_head_kernel,
        out_shape=[
            jax.ShapeDtypeStruct((_KMAX, 68), jnp.float32),
            jax.ShapeDtypeStruct((_KMAX, 2), jnp.float32),
        ],
        scratch_shapes=[pltpu.VMEM((256, 4), jnp.float32)],
    )(sc2, bx1, by1, bx2, by2, packed,
      x1[0].reshape(128, 4096), x2[0].reshape(256, 1024),
      x3[0].reshape(512, 256), x4[0].reshape(1024, 64),
      W1, b1.reshape(1, 64), W2, b2.reshape(1, 64), Wm, bm.reshape(1, 2))
    return out, result, cells
